# Initial kernel scaffold; baseline (speedup 1.0000x reference)
import jax
import jax.numpy as jnp
from jax.experimental import pallas as pl


def kernel(features_fine, features_coarse, point_index_fine, point_index_coarse):
    raise NotImplementedError("write your pallas kernel here")


# speedup vs baseline: 4.1444x; 4.1444x over previous
import functools

import jax
import jax.numpy as jnp
from jax import lax
from jax.experimental import pallas as pl
from jax.experimental.pallas import tpu as pltpu
from jax.experimental.pallas import tpu_sc as plsc

FEAT = 32
D = FEAT + 1
DOUT = 2 * FEAT + 1
CLAMP = 3.0
B = 524288
L = 16
NC = 2
NS = 16
NW = NC * NS
PW = B // NW
C = 512
G = 128
NCHUNK = PW // C


def _sc_body(fine_hbm, coarse_hbm, fidx_hbm, cidx_hbm, out_hbm,
             fidx_v, cidx_v, fidx2_v, cidx2_v, ff_v, cf_v, out_v, fsem, csem):
    wid = lax.axis_index("s") * NC + lax.axis_index("c")
    base_w = wid * PW
    lanes = jnp.arange(L, dtype=jnp.int32)
    col32 = jnp.full((L,), FEAT, jnp.int32)
    col64 = jnp.full((L,), 2 * FEAT, jnp.int32)

    def chunk_body(k, _):
        base = base_w + k * C
        pltpu.sync_copy(fidx_hbm.at[pl.ds(base, C)], fidx_v)
        pltpu.sync_copy(cidx_hbm.at[pl.ds(base, C)], cidx_v)

        for j in range(C // G):
            def clamp_body(g, _, j=j):
                sl = pl.ds(j * G + g * L, L)
                f = jnp.maximum(fidx_v[sl], 0)
                c = jnp.maximum(cidx_v[sl], 0)
                fidx_v[sl] = f
                cidx_v[sl] = c
                fidx2_v[j, pl.ds(g * L, L)] = f
                cidx2_v[j, pl.ds(g * L, L)] = c
                return 0

            lax.fori_loop(0, G // L, clamp_body, 0)

        copies = []
        for j in range(C // G):
            copies.append(pltpu.async_copy(
                fine_hbm.at[fidx2_v.at[j]],
                ff_v.at[pl.ds(j * G, G), :], fsem))
            copies.append(pltpu.async_copy(
                coarse_hbm.at[cidx2_v.at[j]],
                cf_v.at[pl.ds(j * G, G), :], csem))
        for cp in copies:
            cp.wait()

        def grp(g, _):
            rows = g * L + lanes
            sl = pl.ds(g * L, L)
            fi = fidx_v[sl]
            ci = cidx_v[sl]
            ffd = plsc.load_gather(ff_v, [rows, col32])
            cfd = plsc.load_gather(cf_v, [rows, col32])
            fpos = fi > 0
            cpos = ci > 0
            zero = jnp.zeros((L,), jnp.float32)
            ffd = jnp.where(fpos, ffd, zero)
            cfd = jnp.where(cpos, cfd, zero)
            focc = fi >= 0
            cocc = ci >= 0
            dens = jnp.where(focc, ffd, cfd)
            dens = jnp.where(jnp.logical_or(focc, cocc), dens,
                             jnp.full((L,), -CLAMP, jnp.float32))
            plsc.store_scatter(out_v, [rows, col64], dens)

            @pl.when(jnp.min(fi) <= 0)
            def _():
                inval = jnp.logical_not(fpos)
                for c in range(FEAT):
                    plsc.store_scatter(
                        ff_v, [rows, jnp.full((L,), c, jnp.int32)], zero,
                        mask=inval)

            @pl.when(jnp.min(ci) <= 0)
            def _():
                inval = jnp.logical_not(cpos)
                for c in range(FEAT):
                    plsc.store_scatter(
                        cf_v, [rows, jnp.full((L,), c, jnp.int32)], zero,
                        mask=inval)

            return 0

        lax.fori_loop(0, C // L, grp, 0)

        def repack(p, _):
            out_v[p, pl.ds(0, L)] = ff_v[p, pl.ds(0, L)]
            out_v[p, pl.ds(L, L)] = ff_v[p, pl.ds(L, L)]
            out_v[p, pl.ds(2 * L, L)] = cf_v[p, pl.ds(0, L)]
            out_v[p, pl.ds(3 * L, L)] = cf_v[p, pl.ds(L, L)]
            return 0

        lax.fori_loop(0, C, repack, 0)

        pltpu.sync_copy(out_v, out_hbm.at[pl.ds(base, C), :])
        return 0

    lax.fori_loop(0, NCHUNK, chunk_body, 0)


@functools.partial(
    pl.kernel,
    mesh=plsc.VectorSubcoreMesh(core_axis_name="c", subcore_axis_name="s"),
    out_type=jax.ShapeDtypeStruct((B, DOUT), jnp.float32),
    scratch_types=[
        pltpu.VMEM((C,), jnp.int32),
        pltpu.VMEM((C,), jnp.int32),
        pltpu.VMEM((C // G, G), jnp.int32),
        pltpu.VMEM((C // G, G), jnp.int32),
        pltpu.VMEM((C, D), jnp.float32),
        pltpu.VMEM((C, D), jnp.float32),
        pltpu.VMEM((C, DOUT), jnp.float32),
        pltpu.SemaphoreType.DMA,
        pltpu.SemaphoreType.DMA,
    ],
    compiler_params=pltpu.CompilerParams(use_tc_tiling_on_sc=False,
                                         needs_layout_passes=False),
)
def _sc_gather(fine_hbm, coarse_hbm, fidx_hbm, cidx_hbm, out_hbm,
               fidx_v, cidx_v, fidx2_v, cidx2_v, ff_v, cf_v, out_v,
               fsem, csem):
    _sc_body(fine_hbm, coarse_hbm, fidx_hbm, cidx_hbm, out_hbm,
             fidx_v, cidx_v, fidx2_v, cidx2_v, ff_v, cf_v, out_v,
             fsem, csem)


def kernel(features_fine, features_coarse, point_index_fine, point_index_coarse):
    out = _sc_gather(features_fine, features_coarse,
                     point_index_fine.astype(jnp.int32),
                     point_index_coarse.astype(jnp.int32))
    return out[None]
